# baseline (speedup 1.0000x reference)
import jax
import jax.numpy as jnp
from jax import lax
from jax.experimental import pallas as pl

_BATCH, _CW = 64, 4096
_DC, _K, _DE = 64, 1024, 64
_CPB = 8


def _vq_body(x_ref, d_ref, cw_ref, oh_ref):
    for j in range(_CPB):
        xj = x_ref[:, j * _DE:(j + 1) * _DE]
        dj = d_ref[j]
        x_sq = jnp.sum(xj * xj, axis=1, keepdims=True)
        d_sq = jnp.sum(dj * dj, axis=1)[None, :]
        cross = lax.dot_general(xj, dj, (((1,), (1,)), ((), ())),
                                preferred_element_type=jnp.float32)
        dist = x_sq - 2.0 * cross + d_sq
        m = jnp.min(dist, axis=1, keepdims=True)
        ii = lax.broadcasted_iota(jnp.int32, (_BATCH, _K), 1)
        idx = jnp.min(jnp.where(dist == m, ii, _K), axis=1, keepdims=True)
        oh = (ii == idx).astype(jnp.float32)
        oh_ref[:, j, :] = oh
        cw_ref[:, j * _DE:(j + 1) * _DE] = lax.dot_general(
            oh, dj, (((1,), (0,)), ((), ())),
            precision=lax.Precision.HIGHEST,
            preferred_element_type=jnp.float32)


def kernel(x, dictionary):
    cw, oh = pl.pallas_call(
        _vq_body,
        grid=(_DC // _CPB,),
        in_specs=[
            pl.BlockSpec((_BATCH, _CPB * _DE), lambda c: (0, c)),
            pl.BlockSpec((_CPB, _K, _DE), lambda c: (c, 0, 0)),
        ],
        out_specs=[
            pl.BlockSpec((_BATCH, _CPB * _DE), lambda c: (0, c)),
            pl.BlockSpec((_BATCH, _CPB, _K), lambda c: (0, c, 0)),
        ],
        out_shape=[
            jax.ShapeDtypeStruct((_BATCH, _CW), jnp.float32),
            jax.ShapeDtypeStruct((_BATCH, _DC, _K), jnp.float32),
        ],
    )(x, dictionary)
    return cw, oh
